# baseline (speedup 1.0000x reference)
import functools

import jax
import jax.numpy as jnp
from jax import lax
from jax.experimental import pallas as pl
from jax.experimental.pallas import tpu as pltpu
from jax.experimental.pallas import tpu_sc as plsc

EMBEDDING_DIM = 64

_NUM_CORES = 2
_NUM_SUBCORES = 16
_NUM_WORKERS = _NUM_CORES * _NUM_SUBCORES

_CHUNK = 160


@functools.partial(jax.jit, static_argnames=("num_indices",))
def _embedding_gather(weight128, ids_pair, ids_raw, *, num_indices):
    b_per_w = num_indices // _NUM_WORKERS
    n_chunks = b_per_w // _CHUNK
    assert n_chunks % 2 == 0
    mesh = plsc.VectorSubcoreMesh(core_axis_name="c", subcore_axis_name="s")

    @functools.partial(
        pl.kernel,
        mesh=mesh,
        out_type=jax.ShapeDtypeStruct((num_indices, EMBEDDING_DIM), jnp.float32),
        scratch_types=[
            *[pltpu.VMEM((_CHUNK,), jnp.int32) for _ in range(2)],
            *[pltpu.VMEM((_CHUNK,), jnp.int32) for _ in range(2)],
            *[pltpu.VMEM((_CHUNK, 2 * EMBEDDING_DIM), jnp.float32) for _ in range(2)],
            *[pltpu.VMEM((_CHUNK, EMBEDDING_DIM), jnp.float32) for _ in range(2)],
            *[pltpu.SemaphoreType.DMA for _ in range(4)],
        ],
    )
    def gather_kernel(table_hbm, idp_hbm, idr_hbm, out_hbm, *scr):
        idp = scr[0:2]
        idr = scr[2:4]
        pairs = scr[4:6]
        rows = scr[6:8]
        gsem = scr[8:10]
        osem = scr[10:12]

        wid = lax.axis_index("s") * _NUM_CORES + lax.axis_index("c")
        base = wid * b_per_w

        def stage_and_gather(g, b):
            sl = pl.ds(base + g * _CHUNK, _CHUNK)
            pltpu.sync_copy(idp_hbm.at[sl], idp[b])
            pltpu.sync_copy(idr_hbm.at[sl], idr[b])
            return pltpu.async_copy(table_hbm.at[idp[b]], pairs[b], gsem[b])

        def compact(b):
            def cbody(j, carry):
                i0 = j * 16
                colv = (idr[b][pl.ds(i0, 16)] & 1) * EMBEDDING_DIM
                for l in range(16):
                    i = i0 + l
                    col = colv[l]
                    for c in range(EMBEDDING_DIM // 16):
                        rows[b][i, pl.ds(c * 16, 16)] = pairs[b][
                            i, pl.ds(col + c * 16, 16)
                        ]
                return carry

            lax.fori_loop(0, _CHUNK // 16, cbody, 0)

        def start_out(g, b):
            return pltpu.async_copy(
                rows[b], out_hbm.at[pl.ds(base + g * _CHUNK, _CHUNK)], osem[b]
            )

        def wait_gather(b):
            pltpu.make_async_copy(
                table_hbm.at[pl.ds(0, _CHUNK)], pairs[b], gsem[b]
            ).wait()

        def wait_out(b):
            pltpu.make_async_copy(
                out_hbm.at[pl.ds(0, _CHUNK)], rows[b], osem[b]
            ).wait()

        stage_and_gather(0, 0)

        def body(t, carry):
            g0 = 2 * t
            g1 = g0 + 1

            wait_gather(0)
            stage_and_gather(g1, 1)

            @pl.when(t > 0)
            def _():
                wait_out(0)

            compact(0)
            start_out(g0, 0)

            wait_gather(1)

            @pl.when(t + 1 < n_chunks // 2)
            def _():
                stage_and_gather(g1 + 1, 0)

            @pl.when(t > 0)
            def _():
                wait_out(1)

            compact(1)
            start_out(g1, 1)
            return carry

        lax.fori_loop(0, n_chunks // 2, body, 0)
        wait_out(0)
        wait_out(1)

    return gather_kernel(weight128, ids_pair, ids_raw)


def kernel(token_ids, weight):
    batch, seq = token_ids.shape
    num_rows, dim = weight.shape
    flat = token_ids.reshape(-1).astype(jnp.int32)
    w128 = weight.reshape(num_rows // 2, 2 * dim)
    out = _embedding_gather(w128, flat >> 1, flat, num_indices=batch * seq)
    return out.reshape(batch, seq, dim)


# speedup vs baseline: 1.0486x; 1.0486x over previous
import functools

import jax
import jax.numpy as jnp
from jax import lax
from jax.experimental import pallas as pl
from jax.experimental.pallas import tpu as pltpu
from jax.experimental.pallas import tpu_sc as plsc

EMBEDDING_DIM = 64

_NUM_CORES = 2
_NUM_SUBCORES = 16
_NUM_WORKERS = _NUM_CORES * _NUM_SUBCORES

_CHUNK = 512
_CWORDS = _CHUNK * EMBEDDING_DIM


@functools.partial(jax.jit, static_argnames=("num_indices",))
def _embedding_gather(weight_flat, flat_ids, *, num_indices):
    b_per_w = num_indices // _NUM_WORKERS
    n_chunks = b_per_w // _CHUNK
    assert n_chunks % 2 == 0
    mesh = plsc.VectorSubcoreMesh(core_axis_name="c", subcore_axis_name="s")

    @functools.partial(
        pl.kernel,
        mesh=mesh,
        out_type=jax.ShapeDtypeStruct((num_indices * EMBEDDING_DIM,), jnp.float32),
        scratch_types=[
            pltpu.VMEM((b_per_w,), jnp.int32),
            *[pltpu.VMEM((_CWORDS,), jnp.float32) for _ in range(2)],
            *[pltpu.SemaphoreType.DMA for _ in range(4)],
        ],
    )
    def gather_kernel(table_hbm, idx_hbm, out_hbm, idx_v, *scr):
        rows = scr[0:2]
        gsem = scr[2:4]
        osem = scr[4:6]

        wid = lax.axis_index("s") * _NUM_CORES + lax.axis_index("c")
        base = wid * b_per_w

        pltpu.sync_copy(idx_hbm.at[pl.ds(base, b_per_w)], idx_v)

        def issue_gathers(c, b):
            def group(j, carry):
                idv = idx_v[pl.ds(c * _CHUNK + j * 16, 16)]
                woff = idv * EMBEDDING_DIM
                for l in range(16):
                    start = pl.multiple_of(woff[l], EMBEDDING_DIM)
                    pltpu.async_copy(
                        table_hbm.at[pl.ds(start, EMBEDDING_DIM)],
                        rows[b].at[pl.ds((j * 16 + l) * EMBEDDING_DIM, EMBEDDING_DIM)],
                        gsem[b],
                    )
                return carry

            lax.fori_loop(0, _CHUNK // 16, group, 0)

        def drain_gathers(b):
            def one(i, carry):
                pltpu.make_async_copy(
                    table_hbm.at[pl.ds(0, EMBEDDING_DIM)],
                    rows[b].at[pl.ds(0, EMBEDDING_DIM)],
                    gsem[b],
                ).wait()
                return carry

            lax.fori_loop(0, _CHUNK, one, 0)

        def start_out(c, b):
            return pltpu.async_copy(
                rows[b],
                out_hbm.at[pl.ds((base + c * _CHUNK) * EMBEDDING_DIM, _CWORDS)],
                osem[b],
            )

        def wait_out(b):
            pltpu.make_async_copy(
                out_hbm.at[pl.ds(0, _CWORDS)], rows[b], osem[b]
            ).wait()

        def body(t, carry):
            c0 = 2 * t
            c1 = c0 + 1

            @pl.when(t > 0)
            def _():
                wait_out(0)

            issue_gathers(c0, 0)

            @pl.when(t > 0)
            def _():
                wait_out(1)

            issue_gathers(c1, 1)
            drain_gathers(0)
            start_out(c0, 0)
            drain_gathers(1)
            start_out(c1, 1)
            return carry

        lax.fori_loop(0, n_chunks // 2, body, 0)
        wait_out(0)
        wait_out(1)

    return gather_kernel(weight_flat, flat_ids)


def kernel(token_ids, weight):
    batch, seq = token_ids.shape
    dim = weight.shape[1]
    flat = token_ids.reshape(-1).astype(jnp.int32)
    out = _embedding_gather(weight.reshape(-1), flat, num_indices=batch * seq)
    return out.reshape(batch, seq, dim)


# speedup vs baseline: 1.6265x; 1.5511x over previous
import functools

import jax
import jax.numpy as jnp
from jax import lax
from jax.experimental import pallas as pl
from jax.experimental.pallas import tpu as pltpu
from jax.experimental.pallas import tpu_sc as plsc

EMBEDDING_DIM = 64

_NUM_CORES = 2
_NUM_SUBCORES = 16
_NUM_WORKERS = _NUM_CORES * _NUM_SUBCORES

_CHUNK = 320


@functools.partial(jax.jit, static_argnames=("num_indices",))
def _embedding_gather(weight, flat_ids, *, num_indices):
    b_per_w = num_indices // _NUM_WORKERS
    n_chunks = b_per_w // _CHUNK
    assert n_chunks % 2 == 0
    mesh = plsc.VectorSubcoreMesh(core_axis_name="c", subcore_axis_name="s")

    @functools.partial(
        pl.kernel,
        mesh=mesh,
        out_type=jax.ShapeDtypeStruct((num_indices, EMBEDDING_DIM), jnp.float32),
        scratch_types=[
            pltpu.VMEM((b_per_w,), jnp.int32),
            *[pltpu.VMEM((_CHUNK, EMBEDDING_DIM), jnp.float32) for _ in range(2)],
            *[pltpu.SemaphoreType.DMA for _ in range(4)],
        ],
    )
    def gather_kernel(table_hbm, idx_hbm, out_hbm, idx_v, *scr):
        rows = scr[0:2]
        gsem = scr[2:4]
        osem = scr[4:6]

        wid = lax.axis_index("s") * _NUM_CORES + lax.axis_index("c")
        base = wid * b_per_w

        pltpu.sync_copy(idx_hbm.at[pl.ds(base, b_per_w)], idx_v)

        def issue_gathers(c, b):
            def group(j, carry):
                idv = idx_v[pl.ds(c * _CHUNK + j * 16, 16)]
                for l in range(16):
                    pltpu.async_copy(
                        table_hbm.at[pl.ds(idv[l], 1), :],
                        rows[b].at[pl.ds(j * 16 + l, 1), :],
                        gsem[b],
                    )
                return carry

            lax.fori_loop(0, _CHUNK // 16, group, 0)

        def drain_gathers(b):
            def one(i, carry):
                pltpu.make_async_copy(
                    table_hbm.at[pl.ds(0, 1), :],
                    rows[b].at[pl.ds(0, 1), :],
                    gsem[b],
                ).wait()
                return carry

            lax.fori_loop(0, _CHUNK, one, 0)

        def start_out(c, b):
            return pltpu.async_copy(
                rows[b], out_hbm.at[pl.ds(base + c * _CHUNK, _CHUNK)], osem[b]
            )

        def wait_out(b):
            pltpu.make_async_copy(
                out_hbm.at[pl.ds(0, _CHUNK)], rows[b], osem[b]
            ).wait()

        def body(t, carry):
            c0 = 2 * t
            c1 = c0 + 1

            @pl.when(t > 0)
            def _():
                wait_out(0)

            issue_gathers(c0, 0)

            @pl.when(t > 0)
            def _():
                wait_out(1)

            issue_gathers(c1, 1)
            drain_gathers(0)
            start_out(c0, 0)
            drain_gathers(1)
            start_out(c1, 1)
            return carry

        lax.fori_loop(0, n_chunks // 2, body, 0)
        wait_out(0)
        wait_out(1)

    return gather_kernel(weight, flat_ids)


def kernel(token_ids, weight):
    batch, seq = token_ids.shape
    dim = weight.shape[1]
    flat = token_ids.reshape(-1).astype(jnp.int32)
    out = _embedding_gather(weight, flat, num_indices=batch * seq)
    return out.reshape(batch, seq, dim)
